# baseline (speedup 1.0000x reference)
import functools

import jax
import jax.numpy as jnp
import numpy as np
from jax import lax
from jax.experimental import pallas as pl
from jax.experimental.pallas import tpu as pltpu
from jax.experimental.pallas import tpu_sc as plsc

_N = 10000
_E = 320000
_G = 64
_D = 128
_H = 256
_NJ = 20

_W = 128
_CHUNK = 80
_NSUB = 16
_ROWS_PT = 640
_ROWS_LAST = _N - 15 * _ROWS_PT

_IDX_ROWS = 64
_ROWS2D = 4096
_E_PAD = _ROWS2D * _CHUNK
_N_ACC = 10016

_BR = 400
_NBLK = _N // _BR


def _make_edge_agg(rows_per_tile: int, core_row_stride: int):
    n_phases = rows_per_tile // _IDX_ROWS
    mesh = plsc.VectorSubcoreMesh(core_axis_name="c", subcore_axis_name="s")

    @functools.partial(
        pl.kernel,
        out_type=jax.ShapeDtypeStruct((2, _N, _W), jnp.float32),
        mesh=mesh,
        scratch_types=[
            pltpu.VMEM((_IDX_ROWS, _CHUNK), jnp.int32),
            pltpu.VMEM((_IDX_ROWS, _CHUNK), jnp.int32),
            pltpu.VMEM((_CHUNK, _W), jnp.float32),
            pltpu.VMEM((_CHUNK, _W), jnp.float32),
            pltpu.VMEM_SHARED((_N_ACC, _W), jnp.float32),
            pltpu.SemaphoreType.DMA,
        ],
    )
    def agg_kernel(tab_a, tab_b, src2d, dst2d, out, sidx_v, didx_v,
                   rows0, rows1, acc_sh, sem):
        cid = lax.axis_index("c")
        sid = lax.axis_index("s")
        row0 = sid * _ROWS_PT

        zf = jnp.zeros((16,), jnp.float32)

        def _zrow(r, carry):
            for j in range(_W // 16):
                rows0[r, pl.ds(j * 16, 16)] = zf
            return carry

        lax.fori_loop(0, _CHUNK, _zrow, 0)

        @pl.when(sid < _NSUB - 1)
        def _():
            for j in range(_ROWS_PT // _CHUNK):
                pltpu.sync_copy(rows0, acc_sh.at[pl.ds(row0 + j * _CHUNK,
                                                       _CHUNK)])

        @pl.when(sid == _NSUB - 1)
        def _():
            for j in range(_ROWS_LAST // _CHUNK):
                pltpu.sync_copy(rows0, acc_sh.at[pl.ds(row0 + j * _CHUNK,
                                                       _CHUNK)])

        plsc.subcore_barrier()

        def _run_phase(tab, rb):
            pltpu.sync_copy(src2d.at[pl.ds(rb, _IDX_ROWS)], sidx_v)
            pltpu.sync_copy(dst2d.at[pl.ds(rb, _IDX_ROWS)], didx_v)
            pltpu.async_copy(tab.at[sidx_v.at[0]], rows0, sem)

            def _pair(p, carry):
                k0 = 2 * p
                pltpu.make_async_copy(tab.at[sidx_v.at[k0]], rows0,
                                      sem).wait()
                pltpu.async_copy(tab.at[sidx_v.at[k0 + 1]], rows1, sem)
                pltpu.sync_copy(rows0, acc_sh.at[didx_v.at[k0]], add=True)
                pltpu.make_async_copy(tab.at[sidx_v.at[k0 + 1]], rows1,
                                      sem).wait()

                @pl.when(p < _IDX_ROWS // 2 - 1)
                def _():
                    pltpu.async_copy(tab.at[sidx_v.at[k0 + 2]], rows0, sem)

                pltpu.sync_copy(rows1, acc_sh.at[didx_v.at[k0 + 1]],
                                add=True)
                return carry

            lax.fori_loop(0, _IDX_ROWS // 2, _pair, 0)

        base = cid * core_row_stride + sid * rows_per_tile
        for ph in range(n_phases):

            @pl.when(cid == 0)
            def _():
                _run_phase(tab_a, base + ph * _IDX_ROWS)

            @pl.when(cid == 1)
            def _():
                _run_phase(tab_b, base + ph * _IDX_ROWS)

        plsc.subcore_barrier()

        @pl.when(sid < _NSUB - 1)
        def _():
            for j in range(_ROWS_PT // _CHUNK):
                r = row0 + j * _CHUNK
                pltpu.sync_copy(acc_sh.at[pl.ds(r, _CHUNK)], rows0)
                pltpu.sync_copy(rows0, out.at[cid, pl.ds(r, _CHUNK)])

        @pl.when(sid == _NSUB - 1)
        def _():
            for j in range(_ROWS_LAST // _CHUNK):
                r = row0 + j * _CHUNK
                pltpu.sync_copy(acc_sh.at[pl.ds(r, _CHUNK)], rows0)
                pltpu.sync_copy(rows0, out.at[cid, pl.ds(r, _CHUNK)])

    return agg_kernel


_edge_agg_l1 = _make_edge_agg(_ROWS2D // 32, _ROWS2D // 2)
_edge_agg_l2 = _make_edge_agg(_ROWS2D // 16, 0)


def _tc1_body(p_ref, x_ref, wrel_ref, wroot_ref, b_ref, h1a_ref, h1b_ref):
    agg = p_ref[0] + p_ref[1]
    h = jnp.dot(agg, wrel_ref[...], preferred_element_type=jnp.float32)
    h = h + jnp.dot(x_ref[...], wroot_ref[...],
                    preferred_element_type=jnp.float32)
    h = jnp.maximum(h + b_ref[...], 0.0)
    h1a_ref[...] = h[:, :_W]
    h1b_ref[...] = h[:, _W:]


def _tc1(p, x, wrel, wroot, b):
    return pl.pallas_call(
        _tc1_body,
        grid=(_NBLK,),
        in_specs=[
            pl.BlockSpec((2, _BR, _W), lambda i: (0, i, 0)),
            pl.BlockSpec((_BR, _D), lambda i: (i, 0)),
            pl.BlockSpec((_D, _H), lambda i: (0, 0)),
            pl.BlockSpec((_D, _H), lambda i: (0, 0)),
            pl.BlockSpec((1, _H), lambda i: (0, 0)),
        ],
        out_specs=[
            pl.BlockSpec((_BR, _W), lambda i: (i, 0)),
            pl.BlockSpec((_BR, _W), lambda i: (i, 0)),
        ],
        out_shape=[
            jax.ShapeDtypeStruct((_N, _W), jnp.float32),
            jax.ShapeDtypeStruct((_N, _W), jnp.float32),
        ],
    )(p, x, wrel, wroot, b)


def _tc2_body(q_ref, h1a_ref, h1b_ref, wrel_a_ref, wrel_b_ref,
              wroot_a_ref, wroot_b_ref, b_ref, bat_ref,
              sums_ref, cnt_ref, acc, cacc):
    i = pl.program_id(0)

    @pl.when(i == 0)
    def _():
        acc[...] = jnp.zeros_like(acc)
        cacc[...] = jnp.zeros_like(cacc)

    h = jnp.dot(q_ref[0], wrel_a_ref[...], preferred_element_type=jnp.float32)
    h = h + jnp.dot(q_ref[1], wrel_b_ref[...],
                    preferred_element_type=jnp.float32)
    h = h + jnp.dot(h1a_ref[...], wroot_a_ref[...],
                    preferred_element_type=jnp.float32)
    h = h + jnp.dot(h1b_ref[...], wroot_b_ref[...],
                    preferred_element_type=jnp.float32)
    h = jnp.maximum(h + b_ref[...], 0.0)

    gids = lax.broadcasted_iota(jnp.int32, (_G, _BR), 0)
    m = (bat_ref[0] == gids).astype(jnp.float32)
    acc[...] += jnp.dot(m, h, preferred_element_type=jnp.float32)
    cacc[...] += jnp.dot(m, jnp.ones((_BR, _W), jnp.float32),
                         preferred_element_type=jnp.float32)

    @pl.when(i == _NBLK - 1)
    def _():
        sums_ref[...] = acc[...]
        cnt_ref[...] = cacc[...]


def _tc2(q, h1a, h1b, wrel_a, wrel_b, wroot_a, wroot_b, b, bat3):
    return pl.pallas_call(
        _tc2_body,
        grid=(_NBLK,),
        in_specs=[
            pl.BlockSpec((2, _BR, _W), lambda i: (0, i, 0)),
            pl.BlockSpec((_BR, _W), lambda i: (i, 0)),
            pl.BlockSpec((_BR, _W), lambda i: (i, 0)),
            pl.BlockSpec((_W, _H), lambda i: (0, 0)),
            pl.BlockSpec((_W, _H), lambda i: (0, 0)),
            pl.BlockSpec((_W, _H), lambda i: (0, 0)),
            pl.BlockSpec((_W, _H), lambda i: (0, 0)),
            pl.BlockSpec((1, _H), lambda i: (0, 0)),
            pl.BlockSpec((1, 1, _BR), lambda i: (i, 0, 0)),
        ],
        out_specs=[
            pl.BlockSpec((_G, _H), lambda i: (0, 0)),
            pl.BlockSpec((_G, _W), lambda i: (0, 0)),
        ],
        out_shape=[
            jax.ShapeDtypeStruct((_G, _H), jnp.float32),
            jax.ShapeDtypeStruct((_G, _W), jnp.float32),
        ],
        scratch_shapes=[
            pltpu.VMEM((_G, _H), jnp.float32),
            pltpu.VMEM((_G, _W), jnp.float32),
        ],
    )(q, h1a, h1b, wrel_a, wrel_b, wroot_a, wroot_b, b, bat3)


def _tc3_body(s_ref, c_ref, w_ref, b_ref, o_ref):
    cnt = jnp.maximum(c_ref[:, 0:1], 1.0)
    pooled = s_ref[...] / cnt
    z = jnp.dot(pooled, w_ref[...], preferred_element_type=jnp.float32)
    z = z + b_ref[...]
    col = lax.broadcasted_iota(jnp.int32, (_G, 128), 1)
    t = (col - 1) % 5
    zs = jax.nn.sigmoid(z)
    relu_mask = (t == 0) | (t == 3) | (t == 4)
    jets = jnp.where(relu_mask, jnp.maximum(z, 0.0),
                     jnp.where(t == 2, zs * np.float32(2.0 * np.pi), z))
    o_ref[...] = jnp.where(col == 0, zs, jets)


def _tc3(sums, cnt, w, b):
    return pl.pallas_call(
        _tc3_body,
        grid=(1,),
        in_specs=[
            pl.BlockSpec((_G, _H), lambda i: (0, 0)),
            pl.BlockSpec((_G, _W), lambda i: (0, 0)),
            pl.BlockSpec((_H, 128), lambda i: (0, 0)),
            pl.BlockSpec((1, 128), lambda i: (0, 0)),
        ],
        out_specs=pl.BlockSpec((_G, 128), lambda i: (0, 0)),
        out_shape=jax.ShapeDtypeStruct((_G, 128), jnp.float32),
    )(sums, cnt, w, b)


def kernel(x, edge_index, batch, Wrel1, brel1, Wroot1, Wrel2, brel2, Wroot2,
           Wfc, bfc, Wj, bj):
    npad = _E_PAD - _E
    src2d = jnp.concatenate(
        [edge_index[0], jnp.zeros((npad,), jnp.int32)]).reshape(_ROWS2D,
                                                                _CHUNK)
    dst2d = jnp.concatenate(
        [edge_index[1], jnp.full((npad,), _N, jnp.int32)]).reshape(_ROWS2D,
                                                                   _CHUNK)
    p = _edge_agg_l1(x, x, src2d, dst2d)
    h1a, h1b = _tc1(p, x, Wrel1, Wroot1, brel1.reshape(1, _H))
    q = _edge_agg_l2(h1a, h1b, src2d, dst2d)
    sums, cnt = _tc2(
        q, h1a, h1b,
        Wrel2[:_W], Wrel2[_W:], Wroot2[:_W], Wroot2[_W:],
        brel2.reshape(1, _H), batch.reshape(_NBLK, 1, _BR))
    w_head = jnp.pad(jnp.concatenate([Wfc, Wj], axis=1), ((0, 0), (0, 27)))
    b_head = jnp.pad(jnp.concatenate([bfc, bj])[None, :], ((0, 0), (0, 27)))
    zact = _tc3(sums, cnt, w_head, b_head)
    out = zact[:, 0:1]
    jets = zact[:, 1:101].reshape(_G, _NJ, 5)
    return (out, jets)


# speedup vs baseline: 1.0427x; 1.0427x over previous
import functools

import jax
import jax.numpy as jnp
import numpy as np
from jax import lax
from jax.experimental import pallas as pl
from jax.experimental.pallas import tpu as pltpu
from jax.experimental.pallas import tpu_sc as plsc

_N = 10000
_E = 320000
_G = 64
_D = 128
_H = 256
_NJ = 20

_W = 128
_CHUNK = 128
_NSUB = 16
_ROWS_PT = 640
_ROWS_LAST = _N - 15 * _ROWS_PT

_IDX_ROWS = 40
_ROWS2D = 2560
_E_PAD = _ROWS2D * _CHUNK
_N_ACC = 10016

_BR = 400
_NBLK = _N // _BR


def _make_edge_agg(rows_per_tile: int, core_row_stride: int):
    n_phases = rows_per_tile // _IDX_ROWS
    mesh = plsc.VectorSubcoreMesh(core_axis_name="c", subcore_axis_name="s")

    @functools.partial(
        pl.kernel,
        out_type=jax.ShapeDtypeStruct((2, _N, _W), jnp.float32),
        mesh=mesh,
        scratch_types=[
            pltpu.VMEM((_IDX_ROWS, _CHUNK), jnp.int32),
            pltpu.VMEM((_IDX_ROWS, _CHUNK), jnp.int32),
            pltpu.VMEM((_CHUNK, _W), jnp.float32),
            pltpu.VMEM_SHARED((_N_ACC, _W), jnp.float32),
            pltpu.SemaphoreType.DMA,
        ],
    )
    def agg_kernel(tab_a, tab_b, src2d, dst2d, out, sidx_v, didx_v,
                   rows0, acc_sh, sem):
        cid = lax.axis_index("c")
        sid = lax.axis_index("s")
        row0 = sid * _ROWS_PT

        zf = jnp.zeros((16,), jnp.float32)

        def _zrow(r, carry):
            for j in range(_W // 16):
                rows0[r, pl.ds(j * 16, 16)] = zf
            return carry

        lax.fori_loop(0, _CHUNK, _zrow, 0)

        @pl.when(sid < _NSUB - 1)
        def _():
            for j in range(_ROWS_PT // _CHUNK):
                pltpu.sync_copy(rows0, acc_sh.at[pl.ds(row0 + j * _CHUNK,
                                                       _CHUNK)])

        @pl.when(sid == _NSUB - 1)
        def _():
            for j in range(_ROWS_LAST // 80):
                pltpu.sync_copy(rows0.at[pl.ds(0, 80)],
                                acc_sh.at[pl.ds(row0 + j * 80, 80)])

        plsc.subcore_barrier()

        def _run_phase(tab, rb):
            pltpu.sync_copy(src2d.at[pl.ds(rb, _IDX_ROWS)], sidx_v)
            pltpu.sync_copy(dst2d.at[pl.ds(rb, _IDX_ROWS)], didx_v)

            def _step(k, carry):
                pltpu.async_copy(tab.at[sidx_v.at[k]], rows0, sem).wait()
                pltpu.sync_copy(rows0, acc_sh.at[didx_v.at[k]], add=True)
                return carry

            lax.fori_loop(0, _IDX_ROWS, _step, 0)

        base = cid * core_row_stride + sid * rows_per_tile
        for ph in range(n_phases):

            @pl.when(cid == 0)
            def _():
                _run_phase(tab_a, base + ph * _IDX_ROWS)

            @pl.when(cid == 1)
            def _():
                _run_phase(tab_b, base + ph * _IDX_ROWS)

        plsc.subcore_barrier()

        @pl.when(sid < _NSUB - 1)
        def _():
            for j in range(_ROWS_PT // _CHUNK):
                r = row0 + j * _CHUNK
                pltpu.sync_copy(acc_sh.at[pl.ds(r, _CHUNK)], rows0)
                pltpu.sync_copy(rows0, out.at[cid, pl.ds(r, _CHUNK)])

        @pl.when(sid == _NSUB - 1)
        def _():
            for j in range(_ROWS_LAST // 80):
                r = row0 + j * 80
                pltpu.sync_copy(acc_sh.at[pl.ds(r, 80)],
                                rows0.at[pl.ds(0, 80)])
                pltpu.sync_copy(rows0.at[pl.ds(0, 80)],
                                out.at[cid, pl.ds(r, 80)])

    return agg_kernel


_edge_agg_l1 = _make_edge_agg(_ROWS2D // 32, _ROWS2D // 2)
_edge_agg_l2 = _make_edge_agg(_ROWS2D // 16, 0)


def _tc1_body(p_ref, x_ref, wrel_ref, wroot_ref, b_ref, h1a_ref, h1b_ref):
    agg = p_ref[0] + p_ref[1]
    h = jnp.dot(agg, wrel_ref[...], preferred_element_type=jnp.float32)
    h = h + jnp.dot(x_ref[...], wroot_ref[...],
                    preferred_element_type=jnp.float32)
    h = jnp.maximum(h + b_ref[...], 0.0)
    h1a_ref[...] = h[:, :_W]
    h1b_ref[...] = h[:, _W:]


def _tc1(p, x, wrel, wroot, b):
    return pl.pallas_call(
        _tc1_body,
        grid=(_NBLK,),
        in_specs=[
            pl.BlockSpec((2, _BR, _W), lambda i: (0, i, 0)),
            pl.BlockSpec((_BR, _D), lambda i: (i, 0)),
            pl.BlockSpec((_D, _H), lambda i: (0, 0)),
            pl.BlockSpec((_D, _H), lambda i: (0, 0)),
            pl.BlockSpec((1, _H), lambda i: (0, 0)),
        ],
        out_specs=[
            pl.BlockSpec((_BR, _W), lambda i: (i, 0)),
            pl.BlockSpec((_BR, _W), lambda i: (i, 0)),
        ],
        out_shape=[
            jax.ShapeDtypeStruct((_N, _W), jnp.float32),
            jax.ShapeDtypeStruct((_N, _W), jnp.float32),
        ],
    )(p, x, wrel, wroot, b)


def _tc2_body(q_ref, h1a_ref, h1b_ref, wrel_a_ref, wrel_b_ref,
              wroot_a_ref, wroot_b_ref, b_ref, bat_ref,
              sums_ref, cnt_ref, acc, cacc):
    i = pl.program_id(0)

    @pl.when(i == 0)
    def _():
        acc[...] = jnp.zeros_like(acc)
        cacc[...] = jnp.zeros_like(cacc)

    h = jnp.dot(q_ref[0], wrel_a_ref[...], preferred_element_type=jnp.float32)
    h = h + jnp.dot(q_ref[1], wrel_b_ref[...],
                    preferred_element_type=jnp.float32)
    h = h + jnp.dot(h1a_ref[...], wroot_a_ref[...],
                    preferred_element_type=jnp.float32)
    h = h + jnp.dot(h1b_ref[...], wroot_b_ref[...],
                    preferred_element_type=jnp.float32)
    h = jnp.maximum(h + b_ref[...], 0.0)

    gids = lax.broadcasted_iota(jnp.int32, (_G, _BR), 0)
    m = (bat_ref[0] == gids).astype(jnp.float32)
    acc[...] += jnp.dot(m, h, preferred_element_type=jnp.float32)
    cacc[...] += jnp.dot(m, jnp.ones((_BR, _W), jnp.float32),
                         preferred_element_type=jnp.float32)

    @pl.when(i == _NBLK - 1)
    def _():
        sums_ref[...] = acc[...]
        cnt_ref[...] = cacc[...]


def _tc2(q, h1a, h1b, wrel_a, wrel_b, wroot_a, wroot_b, b, bat3):
    return pl.pallas_call(
        _tc2_body,
        grid=(_NBLK,),
        in_specs=[
            pl.BlockSpec((2, _BR, _W), lambda i: (0, i, 0)),
            pl.BlockSpec((_BR, _W), lambda i: (i, 0)),
            pl.BlockSpec((_BR, _W), lambda i: (i, 0)),
            pl.BlockSpec((_W, _H), lambda i: (0, 0)),
            pl.BlockSpec((_W, _H), lambda i: (0, 0)),
            pl.BlockSpec((_W, _H), lambda i: (0, 0)),
            pl.BlockSpec((_W, _H), lambda i: (0, 0)),
            pl.BlockSpec((1, _H), lambda i: (0, 0)),
            pl.BlockSpec((1, 1, _BR), lambda i: (i, 0, 0)),
        ],
        out_specs=[
            pl.BlockSpec((_G, _H), lambda i: (0, 0)),
            pl.BlockSpec((_G, _W), lambda i: (0, 0)),
        ],
        out_shape=[
            jax.ShapeDtypeStruct((_G, _H), jnp.float32),
            jax.ShapeDtypeStruct((_G, _W), jnp.float32),
        ],
        scratch_shapes=[
            pltpu.VMEM((_G, _H), jnp.float32),
            pltpu.VMEM((_G, _W), jnp.float32),
        ],
    )(q, h1a, h1b, wrel_a, wrel_b, wroot_a, wroot_b, b, bat3)


def _tc3_body(s_ref, c_ref, w_ref, b_ref, o_ref):
    cnt = jnp.maximum(c_ref[:, 0:1], 1.0)
    pooled = s_ref[...] / cnt
    z = jnp.dot(pooled, w_ref[...], preferred_element_type=jnp.float32)
    z = z + b_ref[...]
    col = lax.broadcasted_iota(jnp.int32, (_G, 128), 1)
    t = (col - 1) % 5
    zs = jax.nn.sigmoid(z)
    relu_mask = (t == 0) | (t == 3) | (t == 4)
    jets = jnp.where(relu_mask, jnp.maximum(z, 0.0),
                     jnp.where(t == 2, zs * np.float32(2.0 * np.pi), z))
    o_ref[...] = jnp.where(col == 0, zs, jets)


def _tc3(sums, cnt, w, b):
    return pl.pallas_call(
        _tc3_body,
        grid=(1,),
        in_specs=[
            pl.BlockSpec((_G, _H), lambda i: (0, 0)),
            pl.BlockSpec((_G, _W), lambda i: (0, 0)),
            pl.BlockSpec((_H, 128), lambda i: (0, 0)),
            pl.BlockSpec((1, 128), lambda i: (0, 0)),
        ],
        out_specs=pl.BlockSpec((_G, 128), lambda i: (0, 0)),
        out_shape=jax.ShapeDtypeStruct((_G, 128), jnp.float32),
    )(sums, cnt, w, b)


def kernel(x, edge_index, batch, Wrel1, brel1, Wroot1, Wrel2, brel2, Wroot2,
           Wfc, bfc, Wj, bj):
    npad = _E_PAD - _E
    src2d = jnp.concatenate(
        [edge_index[0], jnp.zeros((npad,), jnp.int32)]).reshape(_ROWS2D,
                                                                _CHUNK)
    dst2d = jnp.concatenate(
        [edge_index[1], jnp.full((npad,), _N, jnp.int32)]).reshape(_ROWS2D,
                                                                   _CHUNK)
    p = _edge_agg_l1(x, x, src2d, dst2d)
    h1a, h1b = _tc1(p, x, Wrel1, Wroot1, brel1.reshape(1, _H))
    q = _edge_agg_l2(h1a, h1b, src2d, dst2d)
    sums, cnt = _tc2(
        q, h1a, h1b,
        Wrel2[:_W], Wrel2[_W:], Wroot2[:_W], Wroot2[_W:],
        brel2.reshape(1, _H), batch.reshape(_NBLK, 1, _BR))
    w_head = jnp.pad(jnp.concatenate([Wfc, Wj], axis=1), ((0, 0), (0, 27)))
    b_head = jnp.pad(jnp.concatenate([bfc, bj])[None, :], ((0, 0), (0, 27)))
    zact = _tc3(sums, cnt, w_head, b_head)
    out = zact[:, 0:1]
    jets = zact[:, 1:101].reshape(_G, _NJ, 5)
    return (out, jets)


# speedup vs baseline: 2.1093x; 2.0230x over previous
import functools

import jax
import jax.numpy as jnp
import numpy as np
from jax import lax
from jax.experimental import pallas as pl
from jax.experimental.pallas import tpu as pltpu
from jax.experimental.pallas import tpu_sc as plsc

_N = 10000
_E = 320000
_G = 64
_D = 128
_H = 256
_NJ = 20

_W = 128
_CHUNK = 128
_NSUB = 16
_ROWS_PT = 640
_ROWS_LAST = _N - 15 * _ROWS_PT

_IDX_ROWS = 40
_ROWS2D = 2560
_E_PAD = _ROWS2D * _CHUNK
_N_ACC = 10016

_BR = 400
_NBLK = _N // _BR


def _make_edge_agg(rows_per_tile: int, core_row_stride: int):
    n_phases = rows_per_tile // _IDX_ROWS
    mesh = plsc.VectorSubcoreMesh(core_axis_name="c", subcore_axis_name="s")

    @functools.partial(
        pl.kernel,
        out_type=jax.ShapeDtypeStruct((2, _N, _W), jnp.float32),
        mesh=mesh,
        scratch_types=[
            pltpu.VMEM((_IDX_ROWS, _CHUNK), jnp.int32),
            pltpu.VMEM((_IDX_ROWS, _CHUNK), jnp.int32),
            pltpu.VMEM((_CHUNK, _W), jnp.float32),
            pltpu.VMEM_SHARED((_N_ACC, _W), jnp.float32),
            pltpu.SemaphoreType.DMA,
        ],
    )
    def agg_kernel(tab_a, tab_b, src2d, dst2d, out, sidx_v, didx_v,
                   rows0, acc_sh, sem):
        cid = lax.axis_index("c")
        sid = lax.axis_index("s")
        row0 = sid * _ROWS_PT

        zf = jnp.zeros((16,), jnp.float32)

        def _zrow(r, carry):
            for j in range(_W // 16):
                rows0[r, pl.ds(j * 16, 16)] = zf
            return carry

        lax.fori_loop(0, _CHUNK, _zrow, 0)

        @pl.when(sid < _NSUB - 1)
        def _():
            for j in range(_ROWS_PT // _CHUNK):
                pltpu.sync_copy(rows0, acc_sh.at[pl.ds(row0 + j * _CHUNK,
                                                       _CHUNK)])

        @pl.when(sid == _NSUB - 1)
        def _():
            for j in range(_ROWS_LAST // 80):
                pltpu.sync_copy(rows0.at[pl.ds(0, 80)],
                                acc_sh.at[pl.ds(row0 + j * 80, 80)])

        plsc.subcore_barrier()

        def _run_phase(tab, rb):
            pltpu.sync_copy(src2d.at[pl.ds(rb, _IDX_ROWS)], sidx_v)
            pltpu.sync_copy(dst2d.at[pl.ds(rb, _IDX_ROWS)], didx_v)

            def _step(k, carry):
                pltpu.async_copy(tab.at[sidx_v.at[k]], rows0, sem).wait()
                pltpu.sync_copy(rows0, acc_sh.at[didx_v.at[k]], add=True)
                return carry

            lax.fori_loop(0, _IDX_ROWS, _step, 0)

        base = cid * core_row_stride + sid * rows_per_tile
        for ph in range(n_phases):

            @pl.when(cid == 0)
            def _():
                _run_phase(tab_a, base + ph * _IDX_ROWS)

            @pl.when(cid == 1)
            def _():
                _run_phase(tab_b, base + ph * _IDX_ROWS)

        plsc.subcore_barrier()

        @pl.when(sid < _NSUB - 1)
        def _():
            for j in range(_ROWS_PT // _CHUNK):
                r = row0 + j * _CHUNK
                pltpu.sync_copy(acc_sh.at[pl.ds(r, _CHUNK)], rows0)
                pltpu.sync_copy(rows0, out.at[cid, pl.ds(r, _CHUNK)])

        @pl.when(sid == _NSUB - 1)
        def _():
            for j in range(_ROWS_LAST // 80):
                r = row0 + j * 80
                pltpu.sync_copy(acc_sh.at[pl.ds(r, 80)],
                                rows0.at[pl.ds(0, 80)])
                pltpu.sync_copy(rows0.at[pl.ds(0, 80)],
                                out.at[cid, pl.ds(r, 80)])

    return agg_kernel


_edge_agg_l1 = _make_edge_agg(_ROWS2D // 32, _ROWS2D // 2)
_edge_agg_l2 = _make_edge_agg(_ROWS2D // 16, 0)


def _tc1_body(p_ref, x_ref, wrel_ref, wroot_ref, b_ref, h1a_ref, h1b_ref):
    agg = p_ref[0] + p_ref[1]
    h = jnp.dot(agg, wrel_ref[...], preferred_element_type=jnp.float32)
    h = h + jnp.dot(x_ref[...], wroot_ref[...],
                    preferred_element_type=jnp.float32)
    h = jnp.maximum(h + b_ref[...], 0.0)
    h1a_ref[...] = h[:, :_W]
    h1b_ref[...] = h[:, _W:]


def _tc1(p, x, wrel, wroot, b):
    return pl.pallas_call(
        _tc1_body,
        grid=(_NBLK,),
        in_specs=[
            pl.BlockSpec((2, _BR, _W), lambda i: (0, i, 0)),
            pl.BlockSpec((_BR, _D), lambda i: (i, 0)),
            pl.BlockSpec((_D, _H), lambda i: (0, 0)),
            pl.BlockSpec((_D, _H), lambda i: (0, 0)),
            pl.BlockSpec((1, _H), lambda i: (0, 0)),
        ],
        out_specs=[
            pl.BlockSpec((_BR, _W), lambda i: (i, 0)),
            pl.BlockSpec((_BR, _W), lambda i: (i, 0)),
        ],
        out_shape=[
            jax.ShapeDtypeStruct((_N, _W), jnp.float32),
            jax.ShapeDtypeStruct((_N, _W), jnp.float32),
        ],
    )(p, x, wrel, wroot, b)


def _tc2_body(q_ref, h1a_ref, h1b_ref, wrel_a_ref, wrel_b_ref,
              wroot_a_ref, wroot_b_ref, b_ref, bat_ref,
              sums_ref, cnt_ref, acc, cacc):
    i = pl.program_id(0)

    @pl.when(i == 0)
    def _():
        acc[...] = jnp.zeros_like(acc)
        cacc[...] = jnp.zeros_like(cacc)

    h = jnp.dot(q_ref[0], wrel_a_ref[...], preferred_element_type=jnp.float32)
    h = h + jnp.dot(q_ref[1], wrel_b_ref[...],
                    preferred_element_type=jnp.float32)
    h = h + jnp.dot(h1a_ref[...], wroot_a_ref[...],
                    preferred_element_type=jnp.float32)
    h = h + jnp.dot(h1b_ref[...], wroot_b_ref[...],
                    preferred_element_type=jnp.float32)
    h = jnp.maximum(h + b_ref[...], 0.0)

    gids = lax.broadcasted_iota(jnp.int32, (_G, _BR), 0)
    m = (bat_ref[0] == gids).astype(jnp.float32)
    acc[...] += jnp.dot(m, h, preferred_element_type=jnp.float32)
    cacc[...] += jnp.dot(m, jnp.ones((_BR, _W), jnp.float32),
                         preferred_element_type=jnp.float32)

    @pl.when(i == _NBLK - 1)
    def _():
        sums_ref[...] = acc[...]
        cnt_ref[...] = cacc[...]


def _tc2(q, h1a, h1b, wrel_a, wrel_b, wroot_a, wroot_b, b, bat3):
    return pl.pallas_call(
        _tc2_body,
        grid=(_NBLK,),
        in_specs=[
            pl.BlockSpec((2, _BR, _W), lambda i: (0, i, 0)),
            pl.BlockSpec((_BR, _W), lambda i: (i, 0)),
            pl.BlockSpec((_BR, _W), lambda i: (i, 0)),
            pl.BlockSpec((_W, _H), lambda i: (0, 0)),
            pl.BlockSpec((_W, _H), lambda i: (0, 0)),
            pl.BlockSpec((_W, _H), lambda i: (0, 0)),
            pl.BlockSpec((_W, _H), lambda i: (0, 0)),
            pl.BlockSpec((1, _H), lambda i: (0, 0)),
            pl.BlockSpec((1, 1, _BR), lambda i: (i, 0, 0)),
        ],
        out_specs=[
            pl.BlockSpec((_G, _H), lambda i: (0, 0)),
            pl.BlockSpec((_G, _W), lambda i: (0, 0)),
        ],
        out_shape=[
            jax.ShapeDtypeStruct((_G, _H), jnp.float32),
            jax.ShapeDtypeStruct((_G, _W), jnp.float32),
        ],
        scratch_shapes=[
            pltpu.VMEM((_G, _H), jnp.float32),
            pltpu.VMEM((_G, _W), jnp.float32),
        ],
    )(q, h1a, h1b, wrel_a, wrel_b, wroot_a, wroot_b, b, bat3)


def _tc3_body(s_ref, c_ref, w_ref, b_ref, o_ref):
    cnt = jnp.maximum(c_ref[:, 0:1], 1.0)
    pooled = s_ref[...] / cnt
    z = jnp.dot(pooled, w_ref[...], preferred_element_type=jnp.float32)
    z = z + b_ref[...]
    col = lax.broadcasted_iota(jnp.int32, (_G, 128), 1)
    t = (col - 1) % 5
    zs = jax.nn.sigmoid(z)
    relu_mask = (t == 0) | (t == 3) | (t == 4)
    jets = jnp.where(relu_mask, jnp.maximum(z, 0.0),
                     jnp.where(t == 2, zs * np.float32(2.0 * np.pi), z))
    o_ref[...] = jnp.where(col == 0, zs, jets)


def _tc3(sums, cnt, w, b):
    return pl.pallas_call(
        _tc3_body,
        grid=(1,),
        in_specs=[
            pl.BlockSpec((_G, _H), lambda i: (0, 0)),
            pl.BlockSpec((_G, _W), lambda i: (0, 0)),
            pl.BlockSpec((_H, 128), lambda i: (0, 0)),
            pl.BlockSpec((1, 128), lambda i: (0, 0)),
        ],
        out_specs=pl.BlockSpec((_G, 128), lambda i: (0, 0)),
        out_shape=jax.ShapeDtypeStruct((_G, 128), jnp.float32),
    )(sums, cnt, w, b)


def kernel(x, edge_index, batch, Wrel1, brel1, Wroot1, Wrel2, brel2, Wroot2,
           Wfc, bfc, Wj, bj):
    npad = _E_PAD - _E
    pad_i = jnp.arange(npad, dtype=jnp.int32) % (_N_ACC - _N)
    src2d = jnp.concatenate(
        [edge_index[0], pad_i]).reshape(_ROWS2D, _CHUNK)
    dst2d = jnp.concatenate(
        [edge_index[1], _N + pad_i]).reshape(_ROWS2D, _CHUNK)
    p = _edge_agg_l1(x, x, src2d, dst2d)
    h1a, h1b = _tc1(p, x, Wrel1, Wroot1, brel1.reshape(1, _H))
    q = _edge_agg_l2(h1a, h1b, src2d, dst2d)
    sums, cnt = _tc2(
        q, h1a, h1b,
        Wrel2[:_W], Wrel2[_W:], Wroot2[:_W], Wroot2[_W:],
        brel2.reshape(1, _H), batch.reshape(_NBLK, 1, _BR))
    w_head = jnp.pad(jnp.concatenate([Wfc, Wj], axis=1), ((0, 0), (0, 27)))
    b_head = jnp.pad(jnp.concatenate([bfc, bj])[None, :], ((0, 0), (0, 27)))
    zact = _tc3(sums, cnt, w_head, b_head)
    out = zact[:, 0:1]
    jets = zact[:, 1:101].reshape(_G, _NJ, 5)
    return (out, jets)


# speedup vs baseline: 2.5862x; 1.2260x over previous
import functools

import jax
import jax.numpy as jnp
import numpy as np
from jax import lax
from jax.experimental import pallas as pl
from jax.experimental.pallas import tpu as pltpu
from jax.experimental.pallas import tpu_sc as plsc

_N = 10000
_E = 320000
_G = 64
_D = 128
_H = 256
_NJ = 20

_W = 128
_CHUNK = 128
_NSUB = 16
_ROWS_PT = 640
_ROWS_LAST = _N - 15 * _ROWS_PT

_IDX_ROWS = 16
_ROWS2D = 2560
_E_PAD = _ROWS2D * _CHUNK
_N_ACC = 10016

_BR = 400
_NBLK = _N // _BR


def _make_edge_agg(rows_per_tile: int, core_row_stride: int):
    n_phases = rows_per_tile // _IDX_ROWS
    mesh = plsc.VectorSubcoreMesh(core_axis_name="c", subcore_axis_name="s")

    @functools.partial(
        pl.kernel,
        out_type=jax.ShapeDtypeStruct((2, _N, _W), jnp.float32),
        mesh=mesh,
        scratch_types=[
            pltpu.VMEM((_IDX_ROWS, _CHUNK), jnp.int32),
            pltpu.VMEM((_IDX_ROWS, _CHUNK), jnp.int32),
            pltpu.VMEM((_CHUNK, _W), jnp.float32),
            pltpu.VMEM((_CHUNK, _W), jnp.float32),
            pltpu.VMEM_SHARED((_N_ACC, _W), jnp.float32),
            pltpu.SemaphoreType.DMA,
        ],
    )
    def agg_kernel(tab_a, tab_b, src2d, dst2d, out, sidx_v, didx_v,
                   rows0, rows1, acc_sh, sem):
        cid = lax.axis_index("c")
        sid = lax.axis_index("s")
        row0 = sid * _ROWS_PT

        zf = jnp.zeros((16,), jnp.float32)

        def _zrow(r, carry):
            for j in range(_W // 16):
                rows0[r, pl.ds(j * 16, 16)] = zf
            return carry

        lax.fori_loop(0, _CHUNK, _zrow, 0)

        @pl.when(sid < _NSUB - 1)
        def _():
            for j in range(_ROWS_PT // _CHUNK):
                pltpu.sync_copy(rows0, acc_sh.at[pl.ds(row0 + j * _CHUNK,
                                                       _CHUNK)])

        @pl.when(sid == _NSUB - 1)
        def _():
            for j in range(_ROWS_LAST // 80):
                pltpu.sync_copy(rows0.at[pl.ds(0, 80)],
                                acc_sh.at[pl.ds(row0 + j * 80, 80)])

        plsc.subcore_barrier()

        def _run_phase(tab, rb):
            pltpu.sync_copy(src2d.at[pl.ds(rb, _IDX_ROWS)], sidx_v)
            pltpu.sync_copy(dst2d.at[pl.ds(rb, _IDX_ROWS)], didx_v)
            pltpu.async_copy(tab.at[sidx_v.at[0]], rows0, sem)

            def _pair(p, carry):
                k0 = 2 * p
                pltpu.make_async_copy(tab.at[sidx_v.at[k0]], rows0,
                                      sem).wait()
                pltpu.async_copy(tab.at[sidx_v.at[k0 + 1]], rows1, sem)
                pltpu.sync_copy(rows0, acc_sh.at[didx_v.at[k0]], add=True)
                pltpu.make_async_copy(tab.at[sidx_v.at[k0 + 1]], rows1,
                                      sem).wait()

                @pl.when(p < _IDX_ROWS // 2 - 1)
                def _():
                    pltpu.async_copy(tab.at[sidx_v.at[k0 + 2]], rows0, sem)

                pltpu.sync_copy(rows1, acc_sh.at[didx_v.at[k0 + 1]],
                                add=True)
                return carry

            lax.fori_loop(0, _IDX_ROWS // 2, _pair, 0)

        base = cid * core_row_stride + sid * rows_per_tile
        for ph in range(n_phases):

            @pl.when(cid == 0)
            def _():
                _run_phase(tab_a, base + ph * _IDX_ROWS)

            @pl.when(cid == 1)
            def _():
                _run_phase(tab_b, base + ph * _IDX_ROWS)

        plsc.subcore_barrier()

        @pl.when(sid < _NSUB - 1)
        def _():
            for j in range(_ROWS_PT // _CHUNK):
                r = row0 + j * _CHUNK
                pltpu.sync_copy(acc_sh.at[pl.ds(r, _CHUNK)], rows0)
                pltpu.sync_copy(rows0, out.at[cid, pl.ds(r, _CHUNK)])

        @pl.when(sid == _NSUB - 1)
        def _():
            for j in range(_ROWS_LAST // 80):
                r = row0 + j * 80
                pltpu.sync_copy(acc_sh.at[pl.ds(r, 80)],
                                rows0.at[pl.ds(0, 80)])
                pltpu.sync_copy(rows0.at[pl.ds(0, 80)],
                                out.at[cid, pl.ds(r, 80)])

    return agg_kernel


_edge_agg_l1 = _make_edge_agg(_ROWS2D // 32, _ROWS2D // 2)
_edge_agg_l2 = _make_edge_agg(_ROWS2D // 16, 0)


def _tc1_body(p_ref, x_ref, wrel_ref, wroot_ref, b_ref, h1a_ref, h1b_ref):
    agg = p_ref[0] + p_ref[1]
    h = jnp.dot(agg, wrel_ref[...], preferred_element_type=jnp.float32)
    h = h + jnp.dot(x_ref[...], wroot_ref[...],
                    preferred_element_type=jnp.float32)
    h = jnp.maximum(h + b_ref[...], 0.0)
    h1a_ref[...] = h[:, :_W]
    h1b_ref[...] = h[:, _W:]


def _tc1(p, x, wrel, wroot, b):
    return pl.pallas_call(
        _tc1_body,
        grid=(_NBLK,),
        in_specs=[
            pl.BlockSpec((2, _BR, _W), lambda i: (0, i, 0)),
            pl.BlockSpec((_BR, _D), lambda i: (i, 0)),
            pl.BlockSpec((_D, _H), lambda i: (0, 0)),
            pl.BlockSpec((_D, _H), lambda i: (0, 0)),
            pl.BlockSpec((1, _H), lambda i: (0, 0)),
        ],
        out_specs=[
            pl.BlockSpec((_BR, _W), lambda i: (i, 0)),
            pl.BlockSpec((_BR, _W), lambda i: (i, 0)),
        ],
        out_shape=[
            jax.ShapeDtypeStruct((_N, _W), jnp.float32),
            jax.ShapeDtypeStruct((_N, _W), jnp.float32),
        ],
    )(p, x, wrel, wroot, b)


def _tc2_body(q_ref, h1a_ref, h1b_ref, wrel_a_ref, wrel_b_ref,
              wroot_a_ref, wroot_b_ref, b_ref, bat_ref,
              sums_ref, cnt_ref, acc, cacc):
    i = pl.program_id(0)

    @pl.when(i == 0)
    def _():
        acc[...] = jnp.zeros_like(acc)
        cacc[...] = jnp.zeros_like(cacc)

    h = jnp.dot(q_ref[0], wrel_a_ref[...], preferred_element_type=jnp.float32)
    h = h + jnp.dot(q_ref[1], wrel_b_ref[...],
                    preferred_element_type=jnp.float32)
    h = h + jnp.dot(h1a_ref[...], wroot_a_ref[...],
                    preferred_element_type=jnp.float32)
    h = h + jnp.dot(h1b_ref[...], wroot_b_ref[...],
                    preferred_element_type=jnp.float32)
    h = jnp.maximum(h + b_ref[...], 0.0)

    gids = lax.broadcasted_iota(jnp.int32, (_G, _BR), 0)
    m = (bat_ref[0] == gids).astype(jnp.float32)
    acc[...] += jnp.dot(m, h, preferred_element_type=jnp.float32)
    cacc[...] += jnp.dot(m, jnp.ones((_BR, _W), jnp.float32),
                         preferred_element_type=jnp.float32)

    @pl.when(i == _NBLK - 1)
    def _():
        sums_ref[...] = acc[...]
        cnt_ref[...] = cacc[...]


def _tc2(q, h1a, h1b, wrel_a, wrel_b, wroot_a, wroot_b, b, bat3):
    return pl.pallas_call(
        _tc2_body,
        grid=(_NBLK,),
        in_specs=[
            pl.BlockSpec((2, _BR, _W), lambda i: (0, i, 0)),
            pl.BlockSpec((_BR, _W), lambda i: (i, 0)),
            pl.BlockSpec((_BR, _W), lambda i: (i, 0)),
            pl.BlockSpec((_W, _H), lambda i: (0, 0)),
            pl.BlockSpec((_W, _H), lambda i: (0, 0)),
            pl.BlockSpec((_W, _H), lambda i: (0, 0)),
            pl.BlockSpec((_W, _H), lambda i: (0, 0)),
            pl.BlockSpec((1, _H), lambda i: (0, 0)),
            pl.BlockSpec((1, 1, _BR), lambda i: (i, 0, 0)),
        ],
        out_specs=[
            pl.BlockSpec((_G, _H), lambda i: (0, 0)),
            pl.BlockSpec((_G, _W), lambda i: (0, 0)),
        ],
        out_shape=[
            jax.ShapeDtypeStruct((_G, _H), jnp.float32),
            jax.ShapeDtypeStruct((_G, _W), jnp.float32),
        ],
        scratch_shapes=[
            pltpu.VMEM((_G, _H), jnp.float32),
            pltpu.VMEM((_G, _W), jnp.float32),
        ],
    )(q, h1a, h1b, wrel_a, wrel_b, wroot_a, wroot_b, b, bat3)


def _tc3_body(s_ref, c_ref, w_ref, b_ref, o_ref):
    cnt = jnp.maximum(c_ref[:, 0:1], 1.0)
    pooled = s_ref[...] / cnt
    z = jnp.dot(pooled, w_ref[...], preferred_element_type=jnp.float32)
    z = z + b_ref[...]
    col = lax.broadcasted_iota(jnp.int32, (_G, 128), 1)
    t = (col - 1) % 5
    zs = jax.nn.sigmoid(z)
    relu_mask = (t == 0) | (t == 3) | (t == 4)
    jets = jnp.where(relu_mask, jnp.maximum(z, 0.0),
                     jnp.where(t == 2, zs * np.float32(2.0 * np.pi), z))
    o_ref[...] = jnp.where(col == 0, zs, jets)


def _tc3(sums, cnt, w, b):
    return pl.pallas_call(
        _tc3_body,
        grid=(1,),
        in_specs=[
            pl.BlockSpec((_G, _H), lambda i: (0, 0)),
            pl.BlockSpec((_G, _W), lambda i: (0, 0)),
            pl.BlockSpec((_H, 128), lambda i: (0, 0)),
            pl.BlockSpec((1, 128), lambda i: (0, 0)),
        ],
        out_specs=pl.BlockSpec((_G, 128), lambda i: (0, 0)),
        out_shape=jax.ShapeDtypeStruct((_G, 128), jnp.float32),
    )(sums, cnt, w, b)


def kernel(x, edge_index, batch, Wrel1, brel1, Wroot1, Wrel2, brel2, Wroot2,
           Wfc, bfc, Wj, bj):
    npad = _E_PAD - _E
    pad_i = jnp.arange(npad, dtype=jnp.int32) % (_N_ACC - _N)
    src2d = jnp.concatenate(
        [edge_index[0], pad_i]).reshape(_ROWS2D, _CHUNK)
    dst2d = jnp.concatenate(
        [edge_index[1], _N + pad_i]).reshape(_ROWS2D, _CHUNK)
    p = _edge_agg_l1(x, x, src2d, dst2d)
    h1a, h1b = _tc1(p, x, Wrel1, Wroot1, brel1.reshape(1, _H))
    q = _edge_agg_l2(h1a, h1b, src2d, dst2d)
    sums, cnt = _tc2(
        q, h1a, h1b,
        Wrel2[:_W], Wrel2[_W:], Wroot2[:_W], Wroot2[_W:],
        brel2.reshape(1, _H), batch.reshape(_NBLK, 1, _BR))
    w_head = jnp.pad(jnp.concatenate([Wfc, Wj], axis=1), ((0, 0), (0, 27)))
    b_head = jnp.pad(jnp.concatenate([bfc, bj])[None, :], ((0, 0), (0, 27)))
    zact = _tc3(sums, cnt, w_head, b_head)
    out = zact[:, 0:1]
    jets = zact[:, 1:101].reshape(_G, _NJ, 5)
    return (out, jets)


# speedup vs baseline: 2.7803x; 1.0751x over previous
import functools

import jax
import jax.numpy as jnp
import numpy as np
from jax import lax
from jax.experimental import pallas as pl
from jax.experimental.pallas import tpu as pltpu
from jax.experimental.pallas import tpu_sc as plsc

_N = 10000
_E = 320000
_G = 64
_D = 128
_H = 256
_NJ = 20

_W = 128
_CHUNK = 128
_NSUB = 16
_ROWS_PT = 640
_ROWS_LAST = _N - 15 * _ROWS_PT

_IDX_ROWS = 16
_ROWS2D = 2560
_E_PAD = _ROWS2D * _CHUNK
_N_ACC = 10016

_BR = 400
_NBLK = _N // _BR


def _make_edge_agg(rows_per_tile: int, core_row_stride: int):
    n_phases = rows_per_tile // _IDX_ROWS
    mesh = plsc.VectorSubcoreMesh(core_axis_name="c", subcore_axis_name="s")

    @functools.partial(
        pl.kernel,
        out_type=jax.ShapeDtypeStruct((2, _N, _W), jnp.float32),
        mesh=mesh,
        scratch_types=[
            pltpu.VMEM((_IDX_ROWS, _CHUNK), jnp.int32),
            pltpu.VMEM((_IDX_ROWS, _CHUNK), jnp.int32),
            pltpu.VMEM((_CHUNK, _W), jnp.float32),
            pltpu.VMEM((_CHUNK, _W), jnp.float32),
            pltpu.VMEM_SHARED((_N_ACC, _W), jnp.float32),
            pltpu.SemaphoreType.DMA,
        ],
    )
    def agg_kernel(tab_a, tab_b, src2d, dst2d, out, sidx_v, didx_v,
                   rows0, rows1, acc_sh, sem):
        cid = lax.axis_index("c")
        sid = lax.axis_index("s")
        row0 = sid * _ROWS_PT

        zf = jnp.zeros((16,), jnp.float32)

        def _zrow(r, carry):
            for j in range(_W // 16):
                rows0[r, pl.ds(j * 16, 16)] = zf
            return carry

        lax.fori_loop(0, _CHUNK, _zrow, 0)

        @pl.when(sid < _NSUB - 1)
        def _():
            for j in range(_ROWS_PT // _CHUNK):
                pltpu.async_copy(rows0, acc_sh.at[pl.ds(row0 + j * _CHUNK,
                                                        _CHUNK)], sem)
            for j in range(_ROWS_PT // _CHUNK):
                pltpu.make_async_copy(
                    rows0, acc_sh.at[pl.ds(row0 + j * _CHUNK, _CHUNK)],
                    sem).wait()

        @pl.when(sid == _NSUB - 1)
        def _():
            for j in range(_ROWS_LAST // 80):
                pltpu.async_copy(rows0.at[pl.ds(0, 80)],
                                 acc_sh.at[pl.ds(row0 + j * 80, 80)], sem)
            for j in range(_ROWS_LAST // 80):
                pltpu.make_async_copy(
                    rows0.at[pl.ds(0, 80)],
                    acc_sh.at[pl.ds(row0 + j * 80, 80)], sem).wait()

        plsc.subcore_barrier()

        def _run_phase(tab, rb):
            pltpu.sync_copy(src2d.at[pl.ds(rb, _IDX_ROWS)], sidx_v)
            pltpu.sync_copy(dst2d.at[pl.ds(rb, _IDX_ROWS)], didx_v)
            pltpu.async_copy(tab.at[sidx_v.at[0]], rows0, sem)

            def _pair(p, carry):
                k0 = 2 * p
                pltpu.make_async_copy(tab.at[sidx_v.at[k0]], rows0,
                                      sem).wait()
                pltpu.async_copy(tab.at[sidx_v.at[k0 + 1]], rows1, sem)
                pltpu.sync_copy(rows0, acc_sh.at[didx_v.at[k0]], add=True)
                pltpu.make_async_copy(tab.at[sidx_v.at[k0 + 1]], rows1,
                                      sem).wait()

                @pl.when(p < _IDX_ROWS // 2 - 1)
                def _():
                    pltpu.async_copy(tab.at[sidx_v.at[k0 + 2]], rows0, sem)

                pltpu.sync_copy(rows1, acc_sh.at[didx_v.at[k0 + 1]],
                                add=True)
                return carry

            lax.fori_loop(0, _IDX_ROWS // 2, _pair, 0)

        base = cid * core_row_stride + sid * rows_per_tile
        for ph in range(n_phases):

            @pl.when(cid == 0)
            def _():
                _run_phase(tab_a, base + ph * _IDX_ROWS)

            @pl.when(cid == 1)
            def _():
                _run_phase(tab_b, base + ph * _IDX_ROWS)

        plsc.subcore_barrier()

        def _writeback(piece, npieces):
            bufs = (rows0.at[pl.ds(0, piece)], rows1.at[pl.ds(0, piece)])
            pltpu.sync_copy(acc_sh.at[pl.ds(row0, piece)], bufs[0])
            for j in range(npieces):
                r = row0 + j * piece
                buf, nbuf = bufs[j % 2], bufs[1 - j % 2]
                if j + 1 < npieces:
                    pltpu.async_copy(acc_sh.at[pl.ds(r + piece, piece)],
                                     nbuf, sem)
                pltpu.sync_copy(buf, out.at[cid, pl.ds(r, piece)])
                if j + 1 < npieces:
                    pltpu.make_async_copy(
                        acc_sh.at[pl.ds(r + piece, piece)], nbuf, sem).wait()

        @pl.when(sid < _NSUB - 1)
        def _():
            _writeback(_CHUNK, _ROWS_PT // _CHUNK)

        @pl.when(sid == _NSUB - 1)
        def _():
            _writeback(80, _ROWS_LAST // 80)

    return agg_kernel


_edge_agg_l1 = _make_edge_agg(_ROWS2D // 32, _ROWS2D // 2)
_edge_agg_l2 = _make_edge_agg(_ROWS2D // 16, 0)


def _tc1_body(p_ref, x_ref, wrel_ref, wroot_ref, b_ref, h1a_ref, h1b_ref):
    agg = p_ref[0] + p_ref[1]
    h = jnp.dot(agg, wrel_ref[...], preferred_element_type=jnp.float32)
    h = h + jnp.dot(x_ref[...], wroot_ref[...],
                    preferred_element_type=jnp.float32)
    h = jnp.maximum(h + b_ref[...], 0.0)
    h1a_ref[...] = h[:, :_W]
    h1b_ref[...] = h[:, _W:]


def _tc1(p, x, wrel, wroot, b):
    return pl.pallas_call(
        _tc1_body,
        grid=(_NBLK,),
        in_specs=[
            pl.BlockSpec((2, _BR, _W), lambda i: (0, i, 0)),
            pl.BlockSpec((_BR, _D), lambda i: (i, 0)),
            pl.BlockSpec((_D, _H), lambda i: (0, 0)),
            pl.BlockSpec((_D, _H), lambda i: (0, 0)),
            pl.BlockSpec((1, _H), lambda i: (0, 0)),
        ],
        out_specs=[
            pl.BlockSpec((_BR, _W), lambda i: (i, 0)),
            pl.BlockSpec((_BR, _W), lambda i: (i, 0)),
        ],
        out_shape=[
            jax.ShapeDtypeStruct((_N, _W), jnp.float32),
            jax.ShapeDtypeStruct((_N, _W), jnp.float32),
        ],
    )(p, x, wrel, wroot, b)


def _tc2_body(q_ref, h1a_ref, h1b_ref, wrel_a_ref, wrel_b_ref,
              wroot_a_ref, wroot_b_ref, b_ref, bat_ref, wh_ref, bh_ref,
              zact_ref, acc, cacc):
    i = pl.program_id(0)

    @pl.when(i == 0)
    def _():
        acc[...] = jnp.zeros_like(acc)
        cacc[...] = jnp.zeros_like(cacc)

    h = jnp.dot(q_ref[0], wrel_a_ref[...], preferred_element_type=jnp.float32)
    h = h + jnp.dot(q_ref[1], wrel_b_ref[...],
                    preferred_element_type=jnp.float32)
    h = h + jnp.dot(h1a_ref[...], wroot_a_ref[...],
                    preferred_element_type=jnp.float32)
    h = h + jnp.dot(h1b_ref[...], wroot_b_ref[...],
                    preferred_element_type=jnp.float32)
    h = jnp.maximum(h + b_ref[...], 0.0)

    gids = lax.broadcasted_iota(jnp.int32, (_G, _BR), 0)
    m = (bat_ref[0] == gids).astype(jnp.float32)
    acc[...] += jnp.dot(m, h, preferred_element_type=jnp.float32)
    cacc[...] += jnp.dot(m, jnp.ones((_BR, _W), jnp.float32),
                         preferred_element_type=jnp.float32)

    @pl.when(i == _NBLK - 1)
    def _():
        cnt = jnp.maximum(cacc[:, 0:1], 1.0)
        pooled = acc[...] / cnt
        z = jnp.dot(pooled, wh_ref[...], preferred_element_type=jnp.float32)
        z = z + bh_ref[...]
        col = lax.broadcasted_iota(jnp.int32, (_G, 128), 1)
        t = (col - 1) % 5
        zs = jax.nn.sigmoid(z)
        relu_mask = (t == 0) | (t == 3) | (t == 4)
        jets = jnp.where(relu_mask, jnp.maximum(z, 0.0),
                         jnp.where(t == 2, zs * np.float32(2.0 * np.pi), z))
        zact_ref[...] = jnp.where(col == 0, zs, jets)


def _tc2(q, h1a, h1b, wrel_a, wrel_b, wroot_a, wroot_b, b, bat3, wh, bh):
    return pl.pallas_call(
        _tc2_body,
        grid=(_NBLK,),
        in_specs=[
            pl.BlockSpec((2, _BR, _W), lambda i: (0, i, 0)),
            pl.BlockSpec((_BR, _W), lambda i: (i, 0)),
            pl.BlockSpec((_BR, _W), lambda i: (i, 0)),
            pl.BlockSpec((_W, _H), lambda i: (0, 0)),
            pl.BlockSpec((_W, _H), lambda i: (0, 0)),
            pl.BlockSpec((_W, _H), lambda i: (0, 0)),
            pl.BlockSpec((_W, _H), lambda i: (0, 0)),
            pl.BlockSpec((1, _H), lambda i: (0, 0)),
            pl.BlockSpec((1, 1, _BR), lambda i: (i, 0, 0)),
            pl.BlockSpec((_H, 128), lambda i: (0, 0)),
            pl.BlockSpec((1, 128), lambda i: (0, 0)),
        ],
        out_specs=pl.BlockSpec((_G, 128), lambda i: (0, 0)),
        out_shape=jax.ShapeDtypeStruct((_G, 128), jnp.float32),
        scratch_shapes=[
            pltpu.VMEM((_G, _H), jnp.float32),
            pltpu.VMEM((_G, _W), jnp.float32),
        ],
    )(q, h1a, h1b, wrel_a, wrel_b, wroot_a, wroot_b, b, bat3, wh, bh)


def kernel(x, edge_index, batch, Wrel1, brel1, Wroot1, Wrel2, brel2, Wroot2,
           Wfc, bfc, Wj, bj):
    npad = _E_PAD - _E
    pad_i = jnp.arange(npad, dtype=jnp.int32) % (_N_ACC - _N)
    ei_pad = jnp.concatenate(
        [edge_index, jnp.stack([pad_i, _N + pad_i])], axis=1)
    src2d = ei_pad[0].reshape(_ROWS2D, _CHUNK)
    dst2d = ei_pad[1].reshape(_ROWS2D, _CHUNK)
    p = _edge_agg_l1(x, x, src2d, dst2d)
    h1a, h1b = _tc1(p, x, Wrel1, Wroot1, brel1.reshape(1, _H))
    q = _edge_agg_l2(h1a, h1b, src2d, dst2d)
    w_head = jnp.pad(jnp.concatenate([Wfc, Wj], axis=1), ((0, 0), (0, 27)))
    b_head = jnp.pad(jnp.concatenate([bfc, bj])[None, :], ((0, 0), (0, 27)))
    zact = _tc2(
        q, h1a, h1b,
        Wrel2[:_W], Wrel2[_W:], Wroot2[:_W], Wroot2[_W:],
        brel2.reshape(1, _H), batch.reshape(_NBLK, 1, _BR),
        w_head, b_head)
    out = zact[:, 0:1]
    jets = zact[:, 1:101].reshape(_G, _NJ, 5)
    return (out, jets)
